# baseline (speedup 1.0000x reference)
import jax
import jax.numpy as jnp
from jax import lax
from jax.experimental import pallas as pl
from jax.experimental.pallas import tpu as pltpu
from jax.experimental.pallas import tpu_sc as plsc

N = 10000
D = 128
OUT = 128
E = 320000

NC = 2
NS = 16
NW = NC * NS
LANES = 16

CHUNK = 64
NBUF = 4
NRING = 8
CPT = 160
E_PAD = NW * CPT * CHUNK
N_PAD = 10240
RPT = N_PAD // NS


def _sc_body(idx_hbm, aff_hbm, x_hbm, out_hbm,
             agg_sh, ring, aring, splat, bufs, isems, gsems, ssems):
    cid = lax.axis_index("c")
    sid = lax.axis_index("s")
    wid = sid * NC + cid

    zbuf = bufs[0]

    @pl.loop(0, CHUNK)
    def _zero_rows(r):
        for j in range(D // LANES):
            zbuf[r, pl.ds(j * LANES, LANES)] = jnp.zeros((LANES,), jnp.float32)

    r0 = sid * RPT
    for k in range(RPT // CHUNK):
        pltpu.sync_copy(zbuf, agg_sh.at[pl.ds(r0 + k * CHUNK, CHUNK)])

    plsc.subcore_barrier()

    def fetch_idx(g, r):
        return (pltpu.make_async_copy(idx_hbm.at[wid, g], ring.at[r], isems[r]),
                pltpu.make_async_copy(aff_hbm.at[wid, g], aring.at[r], isems[r]))

    def gather(r, b):
        return pltpu.make_async_copy(x_hbm.at[ring.at[r, 0]], bufs[b], gsems[b])

    def scatter(r, b):
        return pltpu.make_async_copy(bufs[b], agg_sh.at[ring.at[r, 1]], ssems[b])

    for r in range(6):
        for dsc in fetch_idx(r, r):
            dsc.start()
    for b in range(3):
        for dsc in fetch_idx(b, b):
            dsc.wait()
        gather(b, b).start()

    @pl.loop(0, CPT // NRING)
    def _outer(it):
        for rr in range(NRING):
            g = it * NRING + rr
            b = rr % NBUF

            @pl.when(g >= 1)
            def _retire():
                scatter((rr + 7) % NRING, (b + 3) % NBUF).wait()

            @pl.when(g + 6 < CPT)
            def _pref_idx():
                for dsc in fetch_idx(g + 6, (rr + 6) % NRING):
                    dsc.start()

            @pl.when(g + 3 < CPT)
            def _pref_rows():
                for dsc in fetch_idx(0, (rr + 3) % NRING):
                    dsc.wait()
                gather((rr + 3) % NRING, (b + 3) % NBUF).start()

            gather(rr, b).wait()
            rows_v = bufs[b]

            @pl.loop(0, CHUNK // LANES)
            def _splat(e16):
                avec = aring[rr, pl.ds(e16 * LANES, LANES)]
                for l in range(LANES):
                    splat[e16 * LANES + l, :] = jnp.full((LANES,), avec[l],
                                                         jnp.float32)

            @pl.loop(0, CHUNK)
            def _scale(e):
                av = splat[e, :]
                for j in range(D // LANES):
                    sl = pl.ds(j * LANES, LANES)
                    rows_v[e, sl] = rows_v[e, sl] * av

            scatter(rr, b).start(add=True)

    scatter((CPT - 1) % NRING, (CPT - 1) % NBUF).wait()

    plsc.subcore_barrier()

    pltpu.sync_copy(agg_sh.at[pl.ds(r0, RPT)],
                    out_hbm.at[cid, pl.ds(r0, RPT)])


@jax.jit
def _segment_sum_sc(idx, aff, x):
    mesh = plsc.VectorSubcoreMesh(core_axis_name="c", subcore_axis_name="s")
    return pl.kernel(
        _sc_body,
        out_type=jax.ShapeDtypeStruct((NC, N_PAD, D), jnp.float32),
        mesh=mesh,
        scratch_types=[
            pltpu.VMEM_SHARED((N_PAD, D), jnp.float32),
            pltpu.VMEM((NRING, 2, CHUNK), jnp.int32),
            pltpu.VMEM((NRING, CHUNK), jnp.float32),
            pltpu.VMEM((CHUNK, LANES), jnp.float32),
            [pltpu.VMEM((CHUNK, D), jnp.float32) for _ in range(NBUF)],
            [pltpu.SemaphoreType.DMA for _ in range(NRING)],
            [pltpu.SemaphoreType.DMA for _ in range(NBUF)],
            [pltpu.SemaphoreType.DMA for _ in range(NBUF)],
        ],
    )(idx, aff, x)


ROW_BLK = 1000


def _mm_body(x_ref, p0_ref, p1_ref, w1_ref, w2_ref, b_ref, o_ref):
    agg = p0_ref[...] + p1_ref[...]
    acc = jnp.dot(x_ref[...], w1_ref[...], preferred_element_type=jnp.float32)
    acc = acc + jnp.dot(agg, w2_ref[...], preferred_element_type=jnp.float32)
    o_ref[...] = acc + b_ref[...]


@jax.jit
def _concat_linear_tc(x, p0, p1, w1t, w2t, b2d):
    grid = (N // ROW_BLK,)
    return pl.pallas_call(
        _mm_body,
        grid=grid,
        in_specs=[
            pl.BlockSpec((ROW_BLK, D), lambda i: (i, 0)),
            pl.BlockSpec((ROW_BLK, D), lambda i: (i, 0)),
            pl.BlockSpec((ROW_BLK, D), lambda i: (i, 0)),
            pl.BlockSpec((D, OUT), lambda i: (0, 0)),
            pl.BlockSpec((D, OUT), lambda i: (0, 0)),
            pl.BlockSpec((1, OUT), lambda i: (0, 0)),
        ],
        out_specs=pl.BlockSpec((ROW_BLK, OUT), lambda i: (i, 0)),
        out_shape=jax.ShapeDtypeStruct((N, OUT), jnp.float32),
    )(x, p0, p1, w1t, w2t, b2d)


def kernel(x, edge_index, affine, W, b):
    pad = E_PAD - E
    shape3 = (NW, CPT, CHUNK)
    src = jnp.concatenate([edge_index[0], jnp.zeros((pad,), jnp.int32)]).reshape(shape3)
    dst = jnp.concatenate([edge_index[1], jnp.zeros((pad,), jnp.int32)]).reshape(shape3)
    aff = jnp.concatenate([affine, jnp.zeros((pad,), jnp.float32)]).reshape(shape3)
    idx = jnp.stack([src, dst], axis=2)

    partials = _segment_sum_sc(idx, aff, x)

    w1t = W[:, :D].T
    w2t = W[:, D:].T
    b2d = b.reshape(1, OUT)
    return _concat_linear_tc(x, partials[0], partials[1], w1t, w2t, b2d)


# speedup vs baseline: 1.7723x; 1.7723x over previous
import jax
import jax.numpy as jnp
from jax import lax
from jax.experimental import pallas as pl
from jax.experimental.pallas import tpu as pltpu
from jax.experimental.pallas import tpu_sc as plsc

N = 10000
D = 128
OUT = 128
E = 320000

NC = 2
NS = 16
NW = NC * NS
LANES = 16

CHUNK = 80
NBUF = 3
NRING = 6
CPT = 126
E_PAD = NW * CPT * CHUNK
N_PAD = 10240
RPT = N_PAD // NS


def _sc_body(idx_hbm, aff_hbm, x_hbm, out_hbm,
             agg_sh, ring, aring, splat, bufs, isems, gsems, ssems):
    cid = lax.axis_index("c")
    sid = lax.axis_index("s")
    wid = sid * NC + cid

    zbuf = bufs[0]

    @pl.loop(0, CHUNK)
    def _zero_rows(r):
        for j in range(D // LANES):
            zbuf[r, pl.ds(j * LANES, LANES)] = jnp.zeros((LANES,), jnp.float32)

    r0 = sid * RPT
    for k in range(RPT // CHUNK):
        pltpu.sync_copy(zbuf, agg_sh.at[pl.ds(r0 + k * CHUNK, CHUNK)])

    plsc.subcore_barrier()

    def fetch_idx(g, r):
        return (pltpu.make_async_copy(idx_hbm.at[wid, g], ring.at[r], isems[r]),
                pltpu.make_async_copy(aff_hbm.at[wid, g], aring.at[r], isems[r]))

    def gather(r, b):
        return pltpu.make_async_copy(x_hbm.at[ring.at[r, 0]], bufs[b], gsems[b])

    def scatter(r, b):
        return pltpu.make_async_copy(bufs[b], agg_sh.at[ring.at[r, 1]], ssems[b])

    for r in range(4):
        for dsc in fetch_idx(r, r):
            dsc.start()
    for b in range(2):
        for dsc in fetch_idx(b, b):
            dsc.wait()
        gather(b, b).start()

    @pl.loop(0, CPT // NRING)
    def _outer(it):
        for rr in range(NRING):
            g = it * NRING + rr
            b = rr % NBUF

            @pl.when(g >= 1)
            def _retire():
                scatter((rr + 5) % NRING, (b + 2) % NBUF).wait()

            @pl.when(g + 4 < CPT)
            def _pref_idx():
                for dsc in fetch_idx(g + 4, (rr + 4) % NRING):
                    dsc.start()

            @pl.when(g + 2 < CPT)
            def _pref_rows():
                for dsc in fetch_idx(0, (rr + 2) % NRING):
                    dsc.wait()
                gather((rr + 2) % NRING, (b + 2) % NBUF).start()

            gather(rr, b).wait()
            rows_v = bufs[b]

            @pl.loop(0, CHUNK // LANES)
            def _splat(e16):
                avec = aring[rr, pl.ds(e16 * LANES, LANES)]
                for l in range(LANES):
                    splat[e16 * LANES + l, :] = jnp.full((LANES,), avec[l],
                                                         jnp.float32)

            @pl.loop(0, CHUNK)
            def _scale(e):
                av = splat[e, :]
                for j in range(D // LANES):
                    sl = pl.ds(j * LANES, LANES)
                    rows_v[e, sl] = rows_v[e, sl] * av

            scatter(rr, b).start(add=True)

    scatter((CPT - 1) % NRING, (CPT - 1) % NBUF).wait()

    plsc.subcore_barrier()

    pltpu.sync_copy(agg_sh.at[pl.ds(r0, RPT)],
                    out_hbm.at[cid, pl.ds(r0, RPT)])


@jax.jit
def _segment_sum_sc(idx, aff, x):
    mesh = plsc.VectorSubcoreMesh(core_axis_name="c", subcore_axis_name="s")
    return pl.kernel(
        _sc_body,
        out_type=jax.ShapeDtypeStruct((NC, N_PAD, D), jnp.float32),
        mesh=mesh,
        scratch_types=[
            pltpu.VMEM_SHARED((N_PAD, D), jnp.float32),
            pltpu.VMEM((NRING, 2, CHUNK), jnp.int32),
            pltpu.VMEM((NRING, CHUNK), jnp.float32),
            pltpu.VMEM((CHUNK, LANES), jnp.float32),
            [pltpu.VMEM((CHUNK, D), jnp.float32) for _ in range(NBUF)],
            [pltpu.SemaphoreType.DMA for _ in range(NRING)],
            [pltpu.SemaphoreType.DMA for _ in range(NBUF)],
            [pltpu.SemaphoreType.DMA for _ in range(NBUF)],
        ],
    )(idx, aff, x)


ROW_BLK = 1000


def _mm_body(x_ref, p0_ref, p1_ref, w1_ref, w2_ref, b_ref, o_ref):
    agg = p0_ref[...] + p1_ref[...]
    acc = jnp.dot(x_ref[...], w1_ref[...], preferred_element_type=jnp.float32)
    acc = acc + jnp.dot(agg, w2_ref[...], preferred_element_type=jnp.float32)
    o_ref[...] = acc + b_ref[...]


@jax.jit
def _concat_linear_tc(x, p0, p1, w1t, w2t, b2d):
    grid = (N // ROW_BLK,)
    return pl.pallas_call(
        _mm_body,
        grid=grid,
        in_specs=[
            pl.BlockSpec((ROW_BLK, D), lambda i: (i, 0)),
            pl.BlockSpec((ROW_BLK, D), lambda i: (i, 0)),
            pl.BlockSpec((ROW_BLK, D), lambda i: (i, 0)),
            pl.BlockSpec((D, OUT), lambda i: (0, 0)),
            pl.BlockSpec((D, OUT), lambda i: (0, 0)),
            pl.BlockSpec((1, OUT), lambda i: (0, 0)),
        ],
        out_specs=pl.BlockSpec((ROW_BLK, OUT), lambda i: (i, 0)),
        out_shape=jax.ShapeDtypeStruct((N, OUT), jnp.float32),
    )(x, p0, p1, w1t, w2t, b2d)


def kernel(x, edge_index, affine, W, b):
    pad = E_PAD - E
    shape3 = (NW, CPT, CHUNK)
    src = jnp.concatenate([edge_index[0], jnp.zeros((pad,), jnp.int32)]).reshape(shape3)
    dst = jnp.concatenate([edge_index[1], jnp.zeros((pad,), jnp.int32)]).reshape(shape3)
    aff = jnp.concatenate([affine, jnp.zeros((pad,), jnp.float32)]).reshape(shape3)
    idx = jnp.stack([src, dst], axis=2)

    partials = _segment_sum_sc(idx, aff, x)

    w1t = W[:, :D].T
    w2t = W[:, D:].T
    b2d = b.reshape(1, OUT)
    return _concat_linear_tc(x, partials[0], partials[1], w1t, w2t, b2d)
